# Initial kernel scaffold; baseline (speedup 1.0000x reference)
import jax
import jax.numpy as jnp
from jax.experimental import pallas as pl


def kernel(x, values):
    raise NotImplementedError("write your pallas kernel here")


# speedup vs baseline: 288.3617x; 288.3617x over previous
import jax
import jax.numpy as jnp
from jax.experimental import pallas as pl


def _vq_block_kernel(x_ref, v_ref, o_ref):
    x = x_ref[...]
    v = v_ref[...]
    v0 = v[:, 0:1]
    best_d = (x - v0) * (x - v0)
    best_v = jnp.broadcast_to(v0, x.shape)
    for j in range(1, 16):
        vj = v[:, j : j + 1]
        d = (x - vj) * (x - vj)
        take = d < best_d
        best_d = jnp.where(take, d, best_d)
        best_v = jnp.where(take, jnp.broadcast_to(vj, x.shape), best_v)
    o_ref[...] = best_v


def kernel(x, values):
    R, L = x.shape
    H = L // 2
    x8 = x.reshape(R * 2, H)
    v8 = jnp.repeat(values, 2, axis=0)

    BLK = 32768
    grid = (pl.cdiv(H, BLK),)
    out8 = pl.pallas_call(
        _vq_block_kernel,
        out_shape=jax.ShapeDtypeStruct((R * 2, H), x.dtype),
        grid=grid,
        in_specs=[
            pl.BlockSpec((R * 2, BLK), lambda i: (0, i)),
            pl.BlockSpec((R * 2, 16), lambda i: (0, 0)),
        ],
        out_specs=pl.BlockSpec((R * 2, BLK), lambda i: (0, i)),
    )(x8, v8)
    return out8.reshape(R, L)


# speedup vs baseline: 528.8006x; 1.8338x over previous
import jax
import jax.numpy as jnp
from jax.experimental import pallas as pl
from jax.experimental.pallas import tpu as pltpu


def _oddeven_merge_sort_pairs(n):
    pairs = []

    def merge(lo, nn, r):
        step = r * 2
        if step < nn:
            merge(lo, nn, step)
            merge(lo + r, nn, step)
            for i in range(lo + r, lo + nn - r, step):
                pairs.append((i, i + r))
        else:
            pairs.append((lo, lo + r))

    def sort(lo, nn):
        if nn > 1:
            m = nn // 2
            sort(lo, m)
            sort(lo + m, m)
            merge(lo, nn, 1)

    sort(0, n)
    return pairs


_SORT16 = _oddeven_merge_sort_pairs(16)


def _vq_block_kernel(x_ref, v_ref, o_ref):
    x = x_ref[...]
    v = v_ref[...]
    cols = [v[:, j : j + 1] for j in range(16)]
    for i, j in _SORT16:
        a, b = cols[i], cols[j]
        cols[i] = jnp.minimum(a, b)
        cols[j] = jnp.maximum(a, b)
    acc = jnp.broadcast_to(cols[0], x.shape)
    for j in range(15):
        mid = (cols[j] + cols[j + 1]) * 0.5
        acc = jnp.where(x > mid, cols[j + 1], acc)
    o_ref[...] = acc


def kernel(x, values):
    R, L = x.shape
    H = L // 2
    x8 = x.reshape(R * 2, H)
    v8 = jnp.repeat(values, 2, axis=0)

    BLK = 32768
    grid = (pl.cdiv(H, BLK),)
    out8 = pl.pallas_call(
        _vq_block_kernel,
        out_shape=jax.ShapeDtypeStruct((R * 2, H), x.dtype),
        grid=grid,
        in_specs=[
            pl.BlockSpec((R * 2, BLK), lambda i: (0, i)),
            pl.BlockSpec((R * 2, 16), lambda i: (0, 0)),
        ],
        out_specs=pl.BlockSpec((R * 2, BLK), lambda i: (0, i)),
        compiler_params=pltpu.CompilerParams(
            dimension_semantics=("parallel",),
        ),
    )(x8, v8)
    return out8.reshape(R, L)


# speedup vs baseline: 570.9987x; 1.0798x over previous
import jax
import jax.numpy as jnp
from jax.experimental import pallas as pl
from jax.experimental.pallas import tpu as pltpu


def _oddeven_merge_sort_pairs(n):
    pairs = []

    def merge(lo, nn, r):
        step = r * 2
        if step < nn:
            merge(lo, nn, step)
            merge(lo + r, nn, step)
            for i in range(lo + r, lo + nn - r, step):
                pairs.append((i, i + r))
        else:
            pairs.append((lo, lo + r))

    def sort(lo, nn):
        if nn > 1:
            m = nn // 2
            sort(lo, m)
            sort(lo + m, m)
            merge(lo, nn, 1)

    sort(0, n)
    return pairs


_SORT16 = _oddeven_merge_sort_pairs(16)


_CHUNK = 512


def _vq_block_kernel(x_ref, v_ref, o_ref):
    v = v_ref[...]
    cols = [v[:, j : j + 1] for j in range(16)]
    for i, j in _SORT16:
        a, b = cols[i], cols[j]
        cols[i] = jnp.minimum(a, b)
        cols[j] = jnp.maximum(a, b)
    mids = [(cols[j] + cols[j + 1]) * 0.5 for j in range(15)]
    blk = x_ref.shape[1]
    for c in range(0, blk, _CHUNK):
        x = x_ref[:, c : c + _CHUNK]
        acc = jnp.broadcast_to(cols[0], x.shape)
        for j in range(15):
            acc = jnp.where(x > mids[j], cols[j + 1], acc)
        o_ref[:, c : c + _CHUNK] = acc


def kernel(x, values):
    R, L = x.shape
    H = L // 2
    x8 = x.reshape(R * 2, H)
    v8 = jnp.repeat(values, 2, axis=0)

    BLK = 32768
    grid = (pl.cdiv(H, BLK),)
    out8 = pl.pallas_call(
        _vq_block_kernel,
        out_shape=jax.ShapeDtypeStruct((R * 2, H), x.dtype),
        grid=grid,
        in_specs=[
            pl.BlockSpec((R * 2, BLK), lambda i: (0, i)),
            pl.BlockSpec((R * 2, 16), lambda i: (0, 0)),
        ],
        out_specs=pl.BlockSpec((R * 2, BLK), lambda i: (0, i)),
        compiler_params=pltpu.CompilerParams(
            dimension_semantics=("parallel",),
        ),
    )(x8, v8)
    return out8.reshape(R, L)


# speedup vs baseline: 1415.8620x; 2.4796x over previous
import jax
import jax.numpy as jnp
from jax.experimental import pallas as pl
from jax.experimental.pallas import tpu as pltpu


def _oddeven_merge_sort_pairs(n):
    pairs = []

    def merge(lo, nn, r):
        step = r * 2
        if step < nn:
            merge(lo, nn, step)
            merge(lo + r, nn, step)
            for i in range(lo + r, lo + nn - r, step):
                pairs.append((i, i + r))
        else:
            pairs.append((lo, lo + r))

    def sort(lo, nn):
        if nn > 1:
            m = nn // 2
            sort(lo, m)
            sort(lo + m, m)
            merge(lo, nn, 1)

    sort(0, n)
    return pairs


_SORT16 = _oddeven_merge_sort_pairs(16)

_CHUNK = 512


def _vq_block_kernel(x_ref, v_ref, o_ref):
    v = v_ref[...]
    cols = [v[:, j : j + 1] for j in range(16)]
    for i, j in _SORT16:
        a, b = cols[i], cols[j]
        cols[i] = jnp.minimum(a, b)
        cols[j] = jnp.maximum(a, b)
    mids = [(cols[j] + cols[j + 1]) * 0.5 for j in range(15)]
    blk = x_ref.shape[1]
    for c in range(0, blk, _CHUNK):
        x = x_ref[:, c : c + _CHUNK]
        acc = jnp.broadcast_to(cols[0], x.shape)
        for j in range(15):
            acc = jnp.where(x > mids[j], cols[j + 1], acc)
        o_ref[:, c : c + _CHUNK] = acc


def kernel(x, values):
    R, L = x.shape
    BLK = 65536
    grid = (pl.cdiv(L, BLK),)
    out = pl.pallas_call(
        _vq_block_kernel,
        out_shape=jax.ShapeDtypeStruct((R, L), x.dtype),
        grid=grid,
        in_specs=[
            pl.BlockSpec((R, BLK), lambda i: (0, i)),
            pl.BlockSpec((R, 16), lambda i: (0, 0)),
        ],
        out_specs=pl.BlockSpec((R, BLK), lambda i: (0, i)),
        compiler_params=pltpu.CompilerParams(
            dimension_semantics=("parallel",),
        ),
    )(x, values)
    return out
